# baseline (speedup 1.0000x reference)
import functools

import jax
import jax.numpy as jnp
from jax import lax
from jax.experimental import pallas as pl
from jax.experimental.pallas import tpu as pltpu
from jax.experimental.pallas import tpu_sc as plsc

D_MODEL = 64
SCALE = 8.0
NUM_WORKERS = 32
CHUNK = 512


@functools.partial(jax.jit, static_argnames=("b_total",))
def _embed_lookup(x_flat, lut, *, b_total):
    b_per_w = b_total // NUM_WORKERS
    n_chunks = b_per_w // CHUNK
    mesh = plsc.VectorSubcoreMesh(core_axis_name="c", subcore_axis_name="s")

    @functools.partial(
        pl.kernel,
        out_type=jax.ShapeDtypeStruct((b_total, D_MODEL), jnp.float32),
        mesh=mesh,
        scratch_types=[
            pltpu.VMEM((CHUNK,), jnp.int32),
            pltpu.VMEM((CHUNK, D_MODEL), jnp.float32),
            pltpu.SemaphoreType.DMA,
        ],
        compiler_params=pltpu.CompilerParams(use_tc_tiling_on_sc=False),
    )
    def k(x_hbm, lut_hbm, out_hbm, idx_v, rows_v, sem):
        wid = lax.axis_index("s") * 2 + lax.axis_index("c")
        base = wid * b_per_w

        def chunk_body(i, carry):
            off = base + i * CHUNK
            pltpu.sync_copy(x_hbm.at[pl.ds(off, CHUNK)], idx_v)
            pltpu.async_copy(lut_hbm.at[idx_v], rows_v, sem).wait()

            def scale_body(r, c2):
                for j in range(D_MODEL // 16):
                    sl = pl.ds(j * 16, 16)
                    rows_v[r, sl] = rows_v[r, sl] * SCALE
                return c2

            lax.fori_loop(0, CHUNK, scale_body, 0, unroll=4)
            pltpu.sync_copy(rows_v, out_hbm.at[pl.ds(off, CHUNK)])
            return carry

        lax.fori_loop(0, n_chunks, chunk_body, 0)

    return k(x_flat, lut)


def kernel(x, lut):
    b, s = x.shape
    vocab, d = lut.shape
    x_flat = x.reshape(-1).astype(jnp.int32)
    out = _embed_lookup(x_flat, lut, b_total=b * s)
    return out.reshape(b, s, d)


# speedup vs baseline: 1.1197x; 1.1197x over previous
import functools

import jax
import jax.numpy as jnp
from jax import lax
from jax.experimental import pallas as pl
from jax.experimental.pallas import tpu as pltpu
from jax.experimental.pallas import tpu_sc as plsc

D_MODEL = 64
SCALE = 8.0
NUM_WORKERS = 32
CHUNK = 512


@functools.partial(jax.jit, static_argnames=("b_total",))
def _embed_lookup(x_flat, lut, *, b_total):
    b_per_w = b_total // NUM_WORKERS
    n_chunks = b_per_w // CHUNK
    assert n_chunks % 2 == 0 and n_chunks >= 4
    mesh = plsc.VectorSubcoreMesh(core_axis_name="c", subcore_axis_name="s")

    @functools.partial(
        pl.kernel,
        out_type=jax.ShapeDtypeStruct((b_total, D_MODEL), jnp.float32),
        mesh=mesh,
        scratch_types=[
            pltpu.VMEM((CHUNK,), jnp.int32),
            pltpu.VMEM((CHUNK,), jnp.int32),
            pltpu.VMEM((CHUNK, D_MODEL), jnp.float32),
            pltpu.VMEM((CHUNK, D_MODEL), jnp.float32),
            pltpu.SemaphoreType.DMA,
            pltpu.SemaphoreType.DMA,
            pltpu.SemaphoreType.DMA,
            pltpu.SemaphoreType.DMA,
        ],
        compiler_params=pltpu.CompilerParams(use_tc_tiling_on_sc=False),
    )
    def k(x_hbm, lut_hbm, out_hbm, idx0, idx1, rows0, rows1, g0, g1, w0, w1):
        wid = lax.axis_index("s") * 2 + lax.axis_index("c")
        base = wid * b_per_w
        idx = (idx0, idx1)
        rows = (rows0, rows1)
        gsem = (g0, g1)
        wsem = (w0, w1)

        def fetch(i, buf):
            pltpu.sync_copy(x_hbm.at[pl.ds(base + i * CHUNK, CHUNK)], idx[buf])
            pltpu.async_copy(lut_hbm.at[idx[buf]], rows[buf], gsem[buf])

        def finish(i, buf):
            pltpu.make_async_copy(lut_hbm.at[idx[buf]], rows[buf], gsem[buf]).wait()

            def scale_body(r, c2):
                for j in range(D_MODEL // 16):
                    sl = pl.ds(j * 16, 16)
                    rows[buf][r, sl] = rows[buf][r, sl] * SCALE
                return c2

            lax.fori_loop(0, CHUNK, scale_body, 0, unroll=4)
            pltpu.async_copy(
                rows[buf], out_hbm.at[pl.ds(base + i * CHUNK, CHUNK)], wsem[buf]
            )

        def drain(i, buf):
            pltpu.make_async_copy(
                rows[buf], out_hbm.at[pl.ds(base + i * CHUNK, CHUNK)], wsem[buf]
            ).wait()

        fetch(0, 0)
        fetch(1, 1)
        finish(0, 0)

        @pl.loop(1, n_chunks - 1, step=2)
        def _(g):
            for b in range(2):
                i = g + b
                cur = (1 + b) % 2
                oth = 1 - cur
                drain(i - 1, oth)
                fetch(i + 1, oth)
                finish(i, cur)

        drain(n_chunks - 2, 0)
        finish(n_chunks - 1, 1)
        drain(n_chunks - 1, 1)

    return k(x_flat, lut)


def kernel(x, lut):
    b, s = x.shape
    vocab, d = lut.shape
    x_flat = x.reshape(-1).astype(jnp.int32)
    out = _embed_lookup(x_flat, lut, b_total=b * s)
    return out.reshape(b, s, d)


# speedup vs baseline: 1.1492x; 1.0264x over previous
import functools

import jax
import jax.numpy as jnp
from jax import lax
from jax.experimental import pallas as pl
from jax.experimental.pallas import tpu as pltpu
from jax.experimental.pallas import tpu_sc as plsc

D_MODEL = 64
SCALE = 8.0
NUM_WORKERS = 32
CHUNK = 400
NSLOT = 4


@functools.partial(jax.jit, static_argnames=("b_total",))
def _embed_lookup(x_flat, lut, *, b_total):
    b_per_w = b_total // NUM_WORKERS
    n_chunks = b_per_w // CHUNK
    assert n_chunks % NSLOT == 0 and n_chunks >= 2 * NSLOT
    mesh = plsc.VectorSubcoreMesh(core_axis_name="c", subcore_axis_name="s")

    @functools.partial(
        pl.kernel,
        out_type=jax.ShapeDtypeStruct((b_total, D_MODEL), jnp.float32),
        mesh=mesh,
        scratch_types=[
            [pltpu.VMEM((CHUNK,), jnp.int32) for _ in range(NSLOT)],
            [pltpu.VMEM((CHUNK, D_MODEL), jnp.float32) for _ in range(NSLOT)],
            [pltpu.SemaphoreType.DMA for _ in range(NSLOT)],
            [pltpu.SemaphoreType.DMA for _ in range(NSLOT)],
            [pltpu.SemaphoreType.DMA for _ in range(NSLOT)],
        ],
        compiler_params=pltpu.CompilerParams(use_tc_tiling_on_sc=False),
    )
    def k(x_hbm, lut_hbm, out_hbm, idx, rows, isem, gsem, wsem):
        wid = lax.axis_index("s") * 2 + lax.axis_index("c")
        base = wid * b_per_w

        def fetch_idx(i, s):
            pltpu.async_copy(
                x_hbm.at[pl.ds(base + i * CHUNK, CHUNK)], idx[s], isem[s]
            )

        def launch_gather(i, s):
            pltpu.make_async_copy(
                x_hbm.at[pl.ds(base + i * CHUNK, CHUNK)], idx[s], isem[s]
            ).wait()
            pltpu.async_copy(lut_hbm.at[idx[s]], rows[s], gsem[s])

        def drain_write(i, s):
            pltpu.make_async_copy(
                rows[s], out_hbm.at[pl.ds(base + i * CHUNK, CHUNK)], wsem[s]
            ).wait()

        def finish(i, s):
            pltpu.make_async_copy(lut_hbm.at[idx[s]], rows[s], gsem[s]).wait()

            def scale_body(r, c2):
                for j in range(D_MODEL // 16):
                    sl = pl.ds(j * 16, 16)
                    rows[s][r, sl] = rows[s][r, sl] * SCALE
                return c2

            lax.fori_loop(0, CHUNK, scale_body, 0, unroll=4)
            pltpu.async_copy(
                rows[s], out_hbm.at[pl.ds(base + i * CHUNK, CHUNK)], wsem[s]
            )

        fetch_idx(0, 0)
        fetch_idx(1, 1)
        launch_gather(0, 0)

        @pl.loop(0, n_chunks, step=NSLOT)
        def _(g):
            for b in range(NSLOT):
                i = g + b
                s1 = (b + 1) % NSLOT
                s2 = (b + 2) % NSLOT

                @pl.when(i + 2 < n_chunks)
                def _():
                    fetch_idx(i + 2, s2)

                @pl.when(i + 1 < n_chunks)
                def _():
                    @pl.when(i + 1 >= NSLOT)
                    def _():
                        drain_write(i + 1 - NSLOT, s1)

                    launch_gather(i + 1, s1)

                finish(i, b)

        for b in range(NSLOT):
            i = n_chunks - NSLOT + b
            drain_write(i, b)

    return k(x_flat, lut)


def kernel(x, lut):
    b, s = x.shape
    vocab, d = lut.shape
    x_flat = x.reshape(-1).astype(jnp.int32)
    out = _embed_lookup(x_flat, lut, b_total=b * s)
    return out.reshape(b, s, d)
